# baseline (speedup 1.0000x reference)
import functools

import jax
import jax.numpy as jnp
from jax import lax
from jax.experimental import pallas as pl
from jax.experimental.pallas import tpu as pltpu
from jax.experimental.pallas import tpu_sc as plsc

_WLOAD = 6144
_NSTAG = 128


def _build_sc_call(n, num_cores, num_subcores):
    nw = num_cores * num_subcores
    n_groups = n // 8
    gpw = n_groups // nw
    mesh = plsc.VectorSubcoreMesh(core_axis_name="c", subcore_axis_name="s")

    @functools.partial(
        pl.kernel,
        mesh=mesh,
        out_type=jax.ShapeDtypeStruct((n, n), jnp.float32),
        scratch_types=[
            pltpu.VMEM((512, 128), jnp.float32),
            pltpu.SemaphoreType.DMA,
            pltpu.SemaphoreType.DMA,
        ],
    )
    def run(mega_hbm, out_hbm, fs_v, load_sem, row_sem):
        wid = lax.axis_index("s") * num_cores + lax.axis_index("c")
        r16 = wid % 16
        half = wid // 16
        pltpu.async_copy(
            mega_hbm.at[pl.ds(512 * r16, 512)], fs_v, load_sem
        ).wait()
        fs_flat = fs_v.reshape(8, 2 * n)
        descs = []
        for k in range(gpw):
            m_idx = gpw * half + k
            row0 = 8 * r16 + 128 * m_idx
            qp = 128 * (31 - m_idx)
            descs.append(
                pltpu.async_copy(
                    fs_flat.at[:, pl.ds(qp, n)],
                    out_hbm.at[pl.ds(row0, 8)],
                    row_sem,
                )
            )
        for d in descs:
            d.wait()

    return run


def kernel(query_len, key_len, bias_embedding_table):
    n = bias_embedding_table.shape[0]
    rf = bias_embedding_table[:, 0][::-1]
    head = jnp.concatenate(
        [rf[_NSTAG - 1:], jnp.full((n,), rf[n - 1], rf.dtype)]
    )
    p3 = jnp.concatenate([head, rf[0:1], rf[0:_NSTAG - 1]])
    mega = jnp.tile(p3, _NSTAG).reshape(2 * n + 1, _NSTAG)
    info = plsc.get_sparse_core_info()
    run = _build_sc_call(n, info.num_cores, info.num_subcores)
    return run(mega.astype(jnp.float32))


# speedup vs baseline: 1.0191x; 1.0191x over previous
import functools

import jax
import jax.numpy as jnp
from jax import lax
from jax.experimental import pallas as pl
from jax.experimental.pallas import tpu as pltpu
from jax.experimental.pallas import tpu_sc as plsc

_WLOAD = 6144
_NSTAG = 128


def _build_sc_call(n, num_cores, num_subcores):
    nw = num_cores * num_subcores
    n_groups = n // 8
    gpw = n_groups // nw
    mesh = plsc.VectorSubcoreMesh(core_axis_name="c", subcore_axis_name="s")

    @functools.partial(
        pl.kernel,
        mesh=mesh,
        out_type=jax.ShapeDtypeStruct((n, n), jnp.float32),
        scratch_types=[
            pltpu.VMEM((8, _WLOAD // 128, 128), jnp.float32),
            pltpu.SemaphoreType.DMA,
            pltpu.SemaphoreType.DMA,
        ],
    )
    def run(mega_hbm, out_hbm, fs_v, load_sem, row_sem):
        wid = lax.axis_index("s") * num_cores + lax.axis_index("c")
        r16 = wid % 16
        half = wid // 16
        c0p = 16 * (1 - half)
        mega_view = mega_hbm.at[pl.ds(0, 64 * _NSTAG), :].reshape(
            _NSTAG, 64, 128
        )
        pltpu.async_copy(
            mega_view.at[pl.ds(8 * r16, 8), pl.ds(c0p, _WLOAD // 128), :],
            fs_v,
            load_sem,
        ).wait()
        fs_flat = fs_v.reshape(8, _WLOAD)
        descs = []
        for k in range(gpw):
            row0 = 8 * r16 + 128 * (gpw * half + k)
            qp = 128 * (15 - k)
            descs.append(
                pltpu.async_copy(
                    fs_flat.at[:, pl.ds(qp, n)],
                    out_hbm.at[pl.ds(row0, 8)],
                    row_sem,
                )
            )
        for d in descs:
            d.wait()

    return run


def kernel(query_len, key_len, bias_embedding_table):
    n = bias_embedding_table.shape[0]
    rf = bias_embedding_table[:, 0][::-1]
    head = jnp.concatenate(
        [rf[_NSTAG - 1:], jnp.full((n,), rf[n - 1], rf.dtype)]
    )
    p3 = jnp.concatenate([head, rf[0:1], rf[0:_NSTAG - 1]])
    mega = jnp.tile(p3, _NSTAG).reshape(2 * n + 1, _NSTAG)
    info = plsc.get_sparse_core_info()
    run = _build_sc_call(n, info.num_cores, info.num_subcores)
    return run(mega.astype(jnp.float32))


# speedup vs baseline: 1.0591x; 1.0392x over previous
import functools

import jax
import jax.numpy as jnp
from jax import lax
from jax.experimental import pallas as pl
from jax.experimental.pallas import tpu as pltpu
from jax.experimental.pallas import tpu_sc as plsc

_WLOAD = 6144
_NSTAG = 128


def _build_sc_call(n, num_cores, num_subcores):
    nw = num_cores * num_subcores
    n_groups = n // 8
    gpw = n_groups // nw
    mesh = plsc.VectorSubcoreMesh(core_axis_name="c", subcore_axis_name="s")

    @functools.partial(
        pl.kernel,
        mesh=mesh,
        out_type=jax.ShapeDtypeStruct((n, n), jnp.float32),
        scratch_types=[
            pltpu.VMEM((8, _WLOAD // 128, 128), jnp.float32),
            pltpu.SemaphoreType.DMA,
            pltpu.SemaphoreType.DMA,
        ],
    )
    def run(mega_hbm, out_hbm, fs_v, load_sem, row_sem):
        wid = lax.axis_index("s") * num_cores + lax.axis_index("c")
        r16 = wid % 16
        half = wid // 16
        c0p = 16 * (1 - half)
        mega_view = mega_hbm.at[pl.ds(0, 64 * _NSTAG), :].reshape(
            _NSTAG, 64, 128
        )
        pltpu.async_copy(
            mega_view.at[pl.ds(8 * r16, 8), pl.ds(c0p, _WLOAD // 128), :],
            fs_v,
            load_sem,
        ).wait()
        fs_flat = fs_v.reshape(8, _WLOAD)
        descs = []
        for k in range(gpw):
            row0 = 8 * r16 + 128 * (gpw * half + k)
            qp = 128 * (15 - k)
            descs.append(
                pltpu.async_copy(
                    fs_flat.at[:, pl.ds(qp, n)],
                    out_hbm.at[pl.ds(row0, 8)],
                    row_sem,
                )
            )
        for d in descs:
            d.wait()

    return run


def kernel(query_len, key_len, bias_embedding_table):
    n = bias_embedding_table.shape[0]
    rf = bias_embedding_table[:, 0][::-1]
    p3 = jnp.concatenate(
        [rf[_NSTAG - 1:], jnp.full((n + 1,), rf[n - 1], rf.dtype),
         rf[0:_NSTAG - 1]]
    )
    mega = jnp.tile(p3, _NSTAG).reshape(2 * n + 1, _NSTAG)
    info = plsc.get_sparse_core_info()
    run = _build_sc_call(n, info.num_cores, info.num_subcores)
    return run(mega.astype(jnp.float32))
